# Initial kernel scaffold; baseline (speedup 1.0000x reference)
import jax
import jax.numpy as jnp
from jax.experimental import pallas as pl


def kernel(x_pheno, x_gene, edge_index_isa, edge_index_rel, edge_index_rev, edge_label_index, Wl, Wr, b):
    raise NotImplementedError("write your pallas kernel here")


# speedup vs baseline: 2.3809x; 2.3809x over previous
import functools

import jax
import jax.numpy as jnp
from jax import lax
from jax.experimental import pallas as pl
from jax.experimental.pallas import tpu as pltpu
from jax.experimental.pallas import tpu_sc as plsc

N_P = 10000
N_G = 10000
D = 128
E = 320000
EL = 100000
L = 3

NC = 2
NS = 16

R = 10240
RT = R // NS
DUMMY = 10016
CR = 80

CW = 128
CB = 40
EPAD = 327680
CH16 = 160
CH32 = 80

DCH = 25
ELPAD = NC * NS * DCH * CW


@functools.lru_cache(maxsize=None)
def _mesh():
    return plsc.VectorSubcoreMesh(core_axis_name="c", subcore_axis_name="s",
                                  num_cores=NC, num_subcores=NS)


def _prep_edges(e_idx):
    src = e_idx[0].astype(jnp.int32)
    dst = e_idx[1].astype(jnp.int32)
    src = jnp.concatenate([src, jnp.zeros((EPAD - E,), jnp.int32)])
    dst = jnp.concatenate([dst, jnp.full((EPAD - E,), DUMMY, jnp.int32)])
    return src, dst


def _counts_body(d_isa, d_rel, d_rev, riota, z128, out,
                 sh0, sh1, sh2, c0, c1, c2, dstv, riov, sem):
    cid = lax.axis_index("c")
    sid = lax.axis_index("s")
    wid = cid * NS + sid
    shs = (sh0, sh1, sh2)
    cnts = (c0, c1, c2)
    dsts = (d_isa, d_rel, d_rev)
    for a in cnts:
        pltpu.sync_copy(z128.at[pl.ds(0, CR)], a)

    @pl.when(sid == 0)
    def _():
        for a in shs:
            pltpu.sync_copy(z128.at[pl.ds(0, CR)], a)

    pltpu.sync_copy(riota, riov)
    plsc.subcore_barrier()
    ones16 = jnp.ones((16,), jnp.float32)
    lane = lax.iota(jnp.int32, 16)
    for e in range(3):
        pltpu.sync_copy(dsts[e].at[wid], dstv)

        @pl.loop(0, 10240 // 16)
        def _(g):
            idx = dstv[pl.ds(g * 16, 16)]
            hi = lax.shift_right_logical(idx, 7)
            lo = lax.bitwise_and(idx, 127)
            for k in range(16):
                plsc.addupdate_scatter(cnts[e], [hi, lo], ones16,
                                       mask=lane == k)

    for e in range(3):
        pltpu.sync_copy(cnts[e], shs[e].at[riov], add=True)
    plsc.subcore_barrier()

    @pl.when(sid == 0)
    def _():
        for e in range(3):
            pltpu.sync_copy(shs[e], out.at[pl.ds((cid * 3 + e) * CR, CR)])


@functools.lru_cache(maxsize=None)
def _counts_kernel():
    return pl.kernel(
        _counts_body,
        out_type=jax.ShapeDtypeStruct((NC * 3 * CR, D), jnp.float32),
        mesh=_mesh(),
        scratch_types=[
            pltpu.VMEM_SHARED((CR, D), jnp.float32),
            pltpu.VMEM_SHARED((CR, D), jnp.float32),
            pltpu.VMEM_SHARED((CR, D), jnp.float32),
            pltpu.VMEM((CR, D), jnp.float32),
            pltpu.VMEM((CR, D), jnp.float32),
            pltpu.VMEM((CR, D), jnp.float32),
            pltpu.VMEM((EPAD // 32,), jnp.int32),
            pltpu.VMEM((CR,), jnp.int32),
            pltpu.SemaphoreType.DMA,
        ],
        compiler_params=pltpu.CompilerParams(needs_layout_passes=False),
    )


def _seg_edges(x, src_hbm, dst_hbm, nchunks, acc, srcv, dstv, rows, sem):
    for blk in range(nchunks // CB):
        pltpu.sync_copy(src_hbm.at[pl.ds(blk * CB, CB)], srcv)
        pltpu.sync_copy(dst_hbm.at[pl.ds(blk * CB, CB)], dstv)

        @pl.loop(0, CB)
        def _(j):
            pltpu.async_copy(x.at[srcv.at[j]], rows, sem).wait()
            pltpu.sync_copy(rows, acc.at[dstv.at[j]], add=True)


def _sum_ab_body(xp, s_isa, d_isa, s_rel, d_rel, z128, out,
                 acc, srcv, dstv, rows, sem):
    cid = lax.axis_index("c")
    sid = lax.axis_index("s")
    rsh = sid * RT
    pltpu.sync_copy(z128.at[pl.ds(rsh, RT)], acc.at[pl.ds(rsh, RT)])
    plsc.subcore_barrier()

    @pl.when(cid == 0)
    def _():
        _seg_edges(xp, s_isa.at[sid], d_isa.at[sid], CH16,
                   acc, srcv, dstv, rows, sem)

    @pl.when(cid == 1)
    def _():
        _seg_edges(xp, s_rel.at[sid], d_rel.at[sid], CH16,
                   acc, srcv, dstv, rows, sem)

    plsc.subcore_barrier()
    pltpu.sync_copy(acc.at[pl.ds(rsh, RT)], out.at[pl.ds(cid * R + rsh, RT)])


@functools.lru_cache(maxsize=None)
def _sum_ab_kernel():
    return pl.kernel(
        _sum_ab_body,
        out_type=jax.ShapeDtypeStruct((NC * R, D), jnp.float32),
        mesh=_mesh(),
        scratch_types=[
            pltpu.VMEM_SHARED((R, D), jnp.float32),
            pltpu.VMEM((CB, CW), jnp.int32),
            pltpu.VMEM((CB, CW), jnp.int32),
            pltpu.VMEM((CW, D), jnp.float32),
            pltpu.SemaphoreType.DMA,
        ],
    )


def _sum_rev_body(xg, s_rev, d_rev, z128, out, acc, srcv, dstv, rows, sem):
    cid = lax.axis_index("c")
    sid = lax.axis_index("s")
    wid = cid * NS + sid
    rsh = sid * RT
    pltpu.sync_copy(z128.at[pl.ds(rsh, RT)], acc.at[pl.ds(rsh, RT)])
    plsc.subcore_barrier()
    _seg_edges(xg, s_rev.at[wid], d_rev.at[wid], CH32,
               acc, srcv, dstv, rows, sem)
    plsc.subcore_barrier()
    pltpu.sync_copy(acc.at[pl.ds(rsh, RT)], out.at[pl.ds(cid * R + rsh, RT)])


@functools.lru_cache(maxsize=None)
def _sum_rev_kernel():
    return pl.kernel(
        _sum_rev_body,
        out_type=jax.ShapeDtypeStruct((NC * R, D), jnp.float32),
        mesh=_mesh(),
        scratch_types=[
            pltpu.VMEM_SHARED((R, D), jnp.float32),
            pltpu.VMEM((CB, CW), jnp.int32),
            pltpu.VMEM((CB, CW), jnp.int32),
            pltpu.VMEM((CW, D), jnp.float32),
            pltpu.SemaphoreType.DMA,
        ],
    )


BR = 1000


def _update_body(act, sums, srev, cnt, xp, xg, wl, wr, b2, op, og):
    c = jnp.maximum(cnt[:, :3] + cnt[:, 3:], 1.0)
    dot = functools.partial(jnp.dot, preferred_element_type=jnp.float32)
    m_isa = sums[0] / c[:, 0:1]
    m_rel = sums[1] / c[:, 1:2]
    m_rev = (srev[0] + srev[1]) / c[:, 2:3]
    p = (dot(m_isa, wl[0]) + dot(m_rev, wl[2])
         + dot(xp[...], wr[0]) + dot(xp[...], wr[2]) + b2[0])
    g = dot(m_rel, wl[1]) + dot(xg[...], wr[1]) + b2[1]
    if act:
        p = jnp.where(p >= 0, p, 0.01 * p)
        g = jnp.where(g >= 0, g, 0.01 * g)
    op[...] = p
    og[...] = g


def _update(act, sums, srev, cnt, xp, xg, wl, wr, b2):
    grid = N_P // BR
    full = jax.ShapeDtypeStruct((N_P, D), jnp.float32)
    return pl.pallas_call(
        functools.partial(_update_body, act),
        grid=(grid,),
        in_specs=[
            pl.BlockSpec((NC, BR, D), lambda i: (0, i, 0)),
            pl.BlockSpec((NC, BR, D), lambda i: (0, i, 0)),
            pl.BlockSpec((BR, 6), lambda i: (i, 0)),
            pl.BlockSpec((BR, D), lambda i: (i, 0)),
            pl.BlockSpec((BR, D), lambda i: (i, 0)),
            pl.BlockSpec((3, D, D), lambda i: (0, 0, 0)),
            pl.BlockSpec((3, D, D), lambda i: (0, 0, 0)),
            pl.BlockSpec((2, 1, D), lambda i: (0, 0, 0)),
        ],
        out_specs=[pl.BlockSpec((BR, D), lambda i: (i, 0))] * 2,
        out_shape=[full, full],
    )(sums, srev, cnt, xp, xg, wl, wr, b2)


def _decode_gather_body(xp, xg, pidx, gidx, outp, outg, pv, gv, rows, sem):
    cid = lax.axis_index("c")
    sid = lax.axis_index("s")
    wid = cid * NS + sid
    pltpu.sync_copy(pidx.at[wid], pv)
    pltpu.sync_copy(gidx.at[wid], gv)
    base = wid * DCH * CW

    @pl.loop(0, DCH)
    def _(j):
        off = base + j * CW
        pltpu.async_copy(xp.at[pv.at[j]], rows, sem).wait()
        pltpu.sync_copy(rows, outp.at[pl.ds(off, CW)])
        pltpu.async_copy(xg.at[gv.at[j]], rows, sem).wait()
        pltpu.sync_copy(rows, outg.at[pl.ds(off, CW)])


@functools.lru_cache(maxsize=None)
def _decode_gather():
    return pl.kernel(
        _decode_gather_body,
        out_type=[jax.ShapeDtypeStruct((ELPAD, D), jnp.float32),
                  jax.ShapeDtypeStruct((ELPAD, D), jnp.float32)],
        mesh=_mesh(),
        scratch_types=[
            pltpu.VMEM((DCH, CW), jnp.int32),
            pltpu.VMEM((DCH, CW), jnp.int32),
            pltpu.VMEM((CW, D), jnp.float32),
            pltpu.SemaphoreType.DMA,
        ],
    )


DB = 4096


def _decode_dot_body(efp, efg, out):
    s = jnp.sum(efp[...] * efg[...], axis=-1)
    out[...] = 1.0 / (1.0 + jnp.exp(-s))


def _decode_dot(efp, efg):
    return pl.pallas_call(
        _decode_dot_body,
        grid=(ELPAD // DB,),
        in_specs=[
            pl.BlockSpec((DB, D), lambda i: (i, 0)),
            pl.BlockSpec((DB, D), lambda i: (i, 0)),
        ],
        out_specs=pl.BlockSpec((DB,), lambda i: (i,)),
        out_shape=jax.ShapeDtypeStruct((ELPAD,), jnp.float32),
    )(efp, efg)


def kernel(x_pheno, x_gene, edge_index_isa, edge_index_rel, edge_index_rev,
           edge_label_index, Wl, Wr, b):
    f32 = jnp.float32
    s_isa, d_isa = _prep_edges(edge_index_isa)
    s_rel, d_rel = _prep_edges(edge_index_rel)
    s_rev, d_rev = _prep_edges(edge_index_rev)

    riota = jnp.arange(CR, dtype=jnp.int32)
    z128 = jnp.zeros((R, D), f32)

    cnt = _counts_kernel()(d_isa.reshape(32, EPAD // 32),
                           d_rel.reshape(32, EPAD // 32),
                           d_rev.reshape(32, EPAD // 32), riota, z128)
    cnt = cnt.reshape(NC * 3, CR * D).T

    b2 = jnp.stack([b[:, 0] + b[:, 2], b[:, 1]], axis=1)
    b2 = b2[:, :, None, :]

    xp, xg = x_pheno, x_gene
    for l in range(L):
        sums = _sum_ab_kernel()(xp, s_isa.reshape(NS, CH16, CW),
                                d_isa.reshape(NS, CH16, CW),
                                s_rel.reshape(NS, CH16, CW),
                                d_rel.reshape(NS, CH16, CW), z128)
        srev = _sum_rev_kernel()(xg, s_rev.reshape(NC * NS, CH32, CW),
                                 d_rev.reshape(NC * NS, CH32, CW), z128)
        xp, xg = _update(l < L - 1, sums.reshape(NC, R, D),
                         srev.reshape(NC, R, D), cnt, xp, xg,
                         Wl[l], Wr[l], b2[l])

    pidx = edge_label_index[0].astype(jnp.int32)
    gidx = edge_label_index[1].astype(jnp.int32)
    pad = ELPAD - EL
    pidx = jnp.concatenate([pidx, jnp.zeros((pad,), jnp.int32)])
    gidx = jnp.concatenate([gidx, jnp.zeros((pad,), jnp.int32)])
    pidx = pidx.reshape(NC * NS, DCH, CW)
    gidx = gidx.reshape(NC * NS, DCH, CW)

    efp, efg = _decode_gather()(xp, xg, pidx, gidx)
    out = _decode_dot(efp, efg)
    return out[:EL]


# speedup vs baseline: 2.6984x; 1.1334x over previous
import functools

import jax
import jax.numpy as jnp
from jax import lax
from jax.experimental import pallas as pl
from jax.experimental.pallas import tpu as pltpu
from jax.experimental.pallas import tpu_sc as plsc

N_P = 10000
N_G = 10000
D = 128
E = 320000
EL = 100000
L = 3

NC = 2
NS = 16

R = 10240
RT = R // NS
DUMMY = 10016
CR = 80

CW = 128
CB = 40
EPAD = 327680
CH16 = 160
CH32 = 80

DCH = 25
ELPAD = NC * NS * DCH * CW


@functools.lru_cache(maxsize=None)
def _mesh():
    return plsc.VectorSubcoreMesh(core_axis_name="c", subcore_axis_name="s",
                                  num_cores=NC, num_subcores=NS)


def _prep_edges(e_idx):
    src = e_idx[0].astype(jnp.int32)
    dst = e_idx[1].astype(jnp.int32)
    src = jnp.concatenate([src, jnp.zeros((EPAD - E,), jnp.int32)])
    dst = jnp.concatenate([dst, jnp.full((EPAD - E,), DUMMY, jnp.int32)])
    return src, dst


def _counts_body(d_isa, d_rel, d_rev, riota, z128, out,
                 sh0, sh1, sh2, c0, c1, c2, dstv, riov, sem):
    cid = lax.axis_index("c")
    sid = lax.axis_index("s")
    wid = cid * NS + sid
    shs = (sh0, sh1, sh2)
    cnts = (c0, c1, c2)
    dsts = (d_isa, d_rel, d_rev)
    for a in cnts:
        pltpu.sync_copy(z128.at[pl.ds(0, CR)], a)

    @pl.when(sid == 0)
    def _():
        for a in shs:
            pltpu.sync_copy(z128.at[pl.ds(0, CR)], a)

    pltpu.sync_copy(riota, riov)
    plsc.subcore_barrier()
    ones16 = jnp.ones((16,), jnp.float32)
    lane = lax.iota(jnp.int32, 16)
    for e in range(3):
        pltpu.sync_copy(dsts[e].at[wid], dstv)

        @pl.loop(0, 10240 // 16)
        def _(g):
            idx = dstv[pl.ds(g * 16, 16)]
            hi = lax.shift_right_logical(idx, 7)
            lo = lax.bitwise_and(idx, 127)
            for k in range(16):
                plsc.addupdate_scatter(cnts[e], [hi, lo], ones16,
                                       mask=lane == k)

    for e in range(3):
        pltpu.sync_copy(cnts[e], shs[e].at[riov], add=True)
    plsc.subcore_barrier()

    @pl.when(sid == 0)
    def _():
        for e in range(3):
            pltpu.sync_copy(shs[e], out.at[pl.ds((cid * 3 + e) * CR, CR)])


@functools.lru_cache(maxsize=None)
def _counts_kernel():
    return pl.kernel(
        _counts_body,
        out_type=jax.ShapeDtypeStruct((NC * 3 * CR, D), jnp.float32),
        mesh=_mesh(),
        scratch_types=[
            pltpu.VMEM_SHARED((CR, D), jnp.float32),
            pltpu.VMEM_SHARED((CR, D), jnp.float32),
            pltpu.VMEM_SHARED((CR, D), jnp.float32),
            pltpu.VMEM((CR, D), jnp.float32),
            pltpu.VMEM((CR, D), jnp.float32),
            pltpu.VMEM((CR, D), jnp.float32),
            pltpu.VMEM((EPAD // 32,), jnp.int32),
            pltpu.VMEM((CR,), jnp.int32),
            pltpu.SemaphoreType.DMA,
        ],
        compiler_params=pltpu.CompilerParams(needs_layout_passes=False),
    )


def _seg_edges(x, src_hbm, dst_hbm, nchunks, acc, srcv, dstv,
               r0, r1, sem0, sem1):
    for blk in range(nchunks // CB):
        pltpu.sync_copy(src_hbm.at[pl.ds(blk * CB, CB)], srcv)
        pltpu.sync_copy(dst_hbm.at[pl.ds(blk * CB, CB)], dstv)
        pltpu.async_copy(x.at[srcv.at[0]], r0, sem0)

        @pl.loop(0, CB // 2 - 1)
        def _(j2):
            a = j2 * 2
            pltpu.async_copy(x.at[srcv.at[a + 1]], r1, sem1)
            pltpu.make_async_copy(x.at[srcv.at[a]], r0, sem0).wait()
            pltpu.sync_copy(r0, acc.at[dstv.at[a]], add=True)
            pltpu.async_copy(x.at[srcv.at[a + 2]], r0, sem0)
            pltpu.make_async_copy(x.at[srcv.at[a + 1]], r1, sem1).wait()
            pltpu.sync_copy(r1, acc.at[dstv.at[a + 1]], add=True)

        a = CB - 2
        pltpu.async_copy(x.at[srcv.at[a + 1]], r1, sem1)
        pltpu.make_async_copy(x.at[srcv.at[a]], r0, sem0).wait()
        pltpu.sync_copy(r0, acc.at[dstv.at[a]], add=True)
        pltpu.make_async_copy(x.at[srcv.at[a + 1]], r1, sem1).wait()
        pltpu.sync_copy(r1, acc.at[dstv.at[a + 1]], add=True)


def _sum_ab_body(xp, s_isa, d_isa, s_rel, d_rel, z128, out,
                 acc, srcv, dstv, r0, r1, sem0, sem1):
    cid = lax.axis_index("c")
    sid = lax.axis_index("s")
    rsh = sid * RT
    pltpu.sync_copy(z128.at[pl.ds(rsh, RT)], acc.at[pl.ds(rsh, RT)])
    plsc.subcore_barrier()

    @pl.when(cid == 0)
    def _():
        _seg_edges(xp, s_isa.at[sid], d_isa.at[sid], CH16,
                   acc, srcv, dstv, r0, r1, sem0, sem1)

    @pl.when(cid == 1)
    def _():
        _seg_edges(xp, s_rel.at[sid], d_rel.at[sid], CH16,
                   acc, srcv, dstv, r0, r1, sem0, sem1)

    plsc.subcore_barrier()
    pltpu.sync_copy(acc.at[pl.ds(rsh, RT)], out.at[pl.ds(cid * R + rsh, RT)])


@functools.lru_cache(maxsize=None)
def _sum_ab_kernel():
    return pl.kernel(
        _sum_ab_body,
        out_type=jax.ShapeDtypeStruct((NC * R, D), jnp.float32),
        mesh=_mesh(),
        scratch_types=[
            pltpu.VMEM_SHARED((R, D), jnp.float32),
            pltpu.VMEM((CB, CW), jnp.int32),
            pltpu.VMEM((CB, CW), jnp.int32),
            pltpu.VMEM((CW, D), jnp.float32),
            pltpu.VMEM((CW, D), jnp.float32),
            pltpu.SemaphoreType.DMA,
            pltpu.SemaphoreType.DMA,
        ],
    )


def _sum_rev_body(xg, s_rev, d_rev, z128, out,
                  acc, srcv, dstv, r0, r1, sem0, sem1):
    cid = lax.axis_index("c")
    sid = lax.axis_index("s")
    wid = cid * NS + sid
    rsh = sid * RT
    pltpu.sync_copy(z128.at[pl.ds(rsh, RT)], acc.at[pl.ds(rsh, RT)])
    plsc.subcore_barrier()
    _seg_edges(xg, s_rev.at[wid], d_rev.at[wid], CH32,
               acc, srcv, dstv, r0, r1, sem0, sem1)
    plsc.subcore_barrier()
    pltpu.sync_copy(acc.at[pl.ds(rsh, RT)], out.at[pl.ds(cid * R + rsh, RT)])


@functools.lru_cache(maxsize=None)
def _sum_rev_kernel():
    return pl.kernel(
        _sum_rev_body,
        out_type=jax.ShapeDtypeStruct((NC * R, D), jnp.float32),
        mesh=_mesh(),
        scratch_types=[
            pltpu.VMEM_SHARED((R, D), jnp.float32),
            pltpu.VMEM((CB, CW), jnp.int32),
            pltpu.VMEM((CB, CW), jnp.int32),
            pltpu.VMEM((CW, D), jnp.float32),
            pltpu.VMEM((CW, D), jnp.float32),
            pltpu.SemaphoreType.DMA,
            pltpu.SemaphoreType.DMA,
        ],
    )


BR = 1000


def _update_body(act, sums, srev, cnt, xp, xg, wl, wr, b2, op, og):
    c = jnp.maximum(cnt[:, :3] + cnt[:, 3:], 1.0)
    dot = functools.partial(jnp.dot, preferred_element_type=jnp.float32)
    m_isa = sums[0] / c[:, 0:1]
    m_rel = sums[1] / c[:, 1:2]
    m_rev = (srev[0] + srev[1]) / c[:, 2:3]
    p = (dot(m_isa, wl[0]) + dot(m_rev, wl[2])
         + dot(xp[...], wr[0]) + dot(xp[...], wr[2]) + b2[0])
    g = dot(m_rel, wl[1]) + dot(xg[...], wr[1]) + b2[1]
    if act:
        p = jnp.where(p >= 0, p, 0.01 * p)
        g = jnp.where(g >= 0, g, 0.01 * g)
    op[...] = p
    og[...] = g


def _update(act, sums, srev, cnt, xp, xg, wl, wr, b2):
    grid = N_P // BR
    full = jax.ShapeDtypeStruct((N_P, D), jnp.float32)
    return pl.pallas_call(
        functools.partial(_update_body, act),
        grid=(grid,),
        in_specs=[
            pl.BlockSpec((NC, BR, D), lambda i: (0, i, 0)),
            pl.BlockSpec((NC, BR, D), lambda i: (0, i, 0)),
            pl.BlockSpec((BR, 6), lambda i: (i, 0)),
            pl.BlockSpec((BR, D), lambda i: (i, 0)),
            pl.BlockSpec((BR, D), lambda i: (i, 0)),
            pl.BlockSpec((3, D, D), lambda i: (0, 0, 0)),
            pl.BlockSpec((3, D, D), lambda i: (0, 0, 0)),
            pl.BlockSpec((2, 1, D), lambda i: (0, 0, 0)),
        ],
        out_specs=[pl.BlockSpec((BR, D), lambda i: (i, 0))] * 2,
        out_shape=[full, full],
    )(sums, srev, cnt, xp, xg, wl, wr, b2)


def _decode_gather_body(xp, xg, pidx, gidx, outp, outg,
                        pv, gv, rp, rg, semp, semg):
    cid = lax.axis_index("c")
    sid = lax.axis_index("s")
    wid = cid * NS + sid
    pltpu.sync_copy(pidx.at[wid], pv)
    pltpu.sync_copy(gidx.at[wid], gv)
    base = wid * DCH * CW
    pltpu.async_copy(xp.at[pv.at[0]], rp, semp)

    @pl.loop(0, DCH - 1)
    def _(j):
        off = base + j * CW
        pltpu.async_copy(xg.at[gv.at[j]], rg, semg)
        pltpu.make_async_copy(xp.at[pv.at[j]], rp, semp).wait()
        pltpu.sync_copy(rp, outp.at[pl.ds(off, CW)])
        pltpu.async_copy(xp.at[pv.at[j + 1]], rp, semp)
        pltpu.make_async_copy(xg.at[gv.at[j]], rg, semg).wait()
        pltpu.sync_copy(rg, outg.at[pl.ds(off, CW)])

    off = base + (DCH - 1) * CW
    pltpu.async_copy(xg.at[gv.at[DCH - 1]], rg, semg)
    pltpu.make_async_copy(xp.at[pv.at[DCH - 1]], rp, semp).wait()
    pltpu.sync_copy(rp, outp.at[pl.ds(off, CW)])
    pltpu.make_async_copy(xg.at[gv.at[DCH - 1]], rg, semg).wait()
    pltpu.sync_copy(rg, outg.at[pl.ds(off, CW)])


@functools.lru_cache(maxsize=None)
def _decode_gather():
    return pl.kernel(
        _decode_gather_body,
        out_type=[jax.ShapeDtypeStruct((ELPAD, D), jnp.float32),
                  jax.ShapeDtypeStruct((ELPAD, D), jnp.float32)],
        mesh=_mesh(),
        scratch_types=[
            pltpu.VMEM((DCH, CW), jnp.int32),
            pltpu.VMEM((DCH, CW), jnp.int32),
            pltpu.VMEM((CW, D), jnp.float32),
            pltpu.VMEM((CW, D), jnp.float32),
            pltpu.SemaphoreType.DMA,
            pltpu.SemaphoreType.DMA,
        ],
    )


DB = 4096


def _decode_dot_body(efp, efg, out):
    s = jnp.sum(efp[...] * efg[...], axis=-1)
    out[...] = 1.0 / (1.0 + jnp.exp(-s))


def _decode_dot(efp, efg):
    return pl.pallas_call(
        _decode_dot_body,
        grid=(ELPAD // DB,),
        in_specs=[
            pl.BlockSpec((DB, D), lambda i: (i, 0)),
            pl.BlockSpec((DB, D), lambda i: (i, 0)),
        ],
        out_specs=pl.BlockSpec((DB,), lambda i: (i,)),
        out_shape=jax.ShapeDtypeStruct((ELPAD,), jnp.float32),
    )(efp, efg)


def kernel(x_pheno, x_gene, edge_index_isa, edge_index_rel, edge_index_rev,
           edge_label_index, Wl, Wr, b):
    f32 = jnp.float32
    s_isa, d_isa = _prep_edges(edge_index_isa)
    s_rel, d_rel = _prep_edges(edge_index_rel)
    s_rev, d_rev = _prep_edges(edge_index_rev)

    riota = jnp.arange(CR, dtype=jnp.int32)
    z128 = jnp.zeros((R, D), f32)

    cnt = _counts_kernel()(d_isa.reshape(32, EPAD // 32),
                           d_rel.reshape(32, EPAD // 32),
                           d_rev.reshape(32, EPAD // 32), riota, z128)
    cnt = cnt.reshape(NC * 3, CR * D).T

    b2 = jnp.stack([b[:, 0] + b[:, 2], b[:, 1]], axis=1)
    b2 = b2[:, :, None, :]

    xp, xg = x_pheno, x_gene
    for l in range(L):
        sums = _sum_ab_kernel()(xp, s_isa.reshape(NS, CH16, CW),
                                d_isa.reshape(NS, CH16, CW),
                                s_rel.reshape(NS, CH16, CW),
                                d_rel.reshape(NS, CH16, CW), z128)
        srev = _sum_rev_kernel()(xg, s_rev.reshape(NC * NS, CH32, CW),
                                 d_rev.reshape(NC * NS, CH32, CW), z128)
        xp, xg = _update(l < L - 1, sums.reshape(NC, R, D),
                         srev.reshape(NC, R, D), cnt, xp, xg,
                         Wl[l], Wr[l], b2[l])

    pidx = edge_label_index[0].astype(jnp.int32)
    gidx = edge_label_index[1].astype(jnp.int32)
    pad = ELPAD - EL
    pidx = jnp.concatenate([pidx, jnp.zeros((pad,), jnp.int32)])
    gidx = jnp.concatenate([gidx, jnp.zeros((pad,), jnp.int32)])
    pidx = pidx.reshape(NC * NS, DCH, CW)
    gidx = gidx.reshape(NC * NS, DCH, CW)

    efp, efg = _decode_gather()(xp, xg, pidx, gidx)
    out = _decode_dot(efp, efg)
    return out[:EL]


# speedup vs baseline: 5.3479x; 1.9819x over previous
import functools

import jax
import jax.numpy as jnp
from jax import lax
from jax.experimental import pallas as pl
from jax.experimental.pallas import tpu as pltpu
from jax.experimental.pallas import tpu_sc as plsc

N_P = 10000
N_G = 10000
D = 128
HD = 64
E = 320000
EL = 100000
L = 3

NC = 2
NS = 16

R = 10240
CR = 80
DUMMY = 10016
RX = 10112
RT = RX // NS

CW = 128
CB = 40
EPAD = 327680
CH16 = 160

DCH = 50
ELPAD = NS * DCH * CW
_SC_PARAMS = None


@functools.lru_cache(maxsize=None)
def _mesh():
    return plsc.VectorSubcoreMesh(core_axis_name="c", subcore_axis_name="s",
                                  num_cores=NC, num_subcores=NS)


def _prep_edges(e_idx):
    src = e_idx[0].astype(jnp.int32)
    dst = e_idx[1].astype(jnp.int32)
    src = jnp.concatenate([src, jnp.zeros((EPAD - E,), jnp.int32)])
    dst = jnp.concatenate([dst, jnp.full((EPAD - E,), DUMMY, jnp.int32)])
    return src.reshape(NS, CH16, CW), dst.reshape(NS, CH16, CW)


def _counts_body(d_isa, d_rel, d_rev, riota, z128, out,
                 sh0, sh1, sh2, c0, c1, c2, dstv, riov, sem):
    cid = lax.axis_index("c")
    sid = lax.axis_index("s")
    wid = cid * NS + sid
    shs = (sh0, sh1, sh2)
    cnts = (c0, c1, c2)
    dsts = (d_isa, d_rel, d_rev)
    for a in cnts:
        pltpu.sync_copy(z128.at[pl.ds(0, CR)], a)

    @pl.when(sid == 0)
    def _():
        for a in shs:
            pltpu.sync_copy(z128.at[pl.ds(0, CR)], a)

    pltpu.sync_copy(riota, riov)
    plsc.subcore_barrier()
    ones16 = jnp.ones((16,), jnp.float32)
    lane = lax.iota(jnp.int32, 16)
    for e in range(3):
        pltpu.sync_copy(dsts[e].at[wid], dstv)

        @pl.loop(0, R // 16)
        def _(g):
            idx = dstv[pl.ds(g * 16, 16)]
            hi = lax.shift_right_logical(idx, 7)
            lo = lax.bitwise_and(idx, 127)
            for k in range(16):
                plsc.addupdate_scatter(cnts[e], [hi, lo], ones16,
                                       mask=lane == k)

    for e in range(3):
        pltpu.sync_copy(cnts[e], shs[e].at[riov], add=True)
    plsc.subcore_barrier()

    @pl.when(sid == 0)
    def _():
        for e in range(3):
            pltpu.sync_copy(shs[e], out.at[pl.ds((cid * 3 + e) * CR, CR)])


@functools.lru_cache(maxsize=None)
def _counts_kernel():
    return pl.kernel(
        _counts_body,
        out_type=jax.ShapeDtypeStruct((NC * 3 * CR, D), jnp.float32),
        mesh=_mesh(),
        scratch_types=[
            pltpu.VMEM_SHARED((CR, D), jnp.float32),
            pltpu.VMEM_SHARED((CR, D), jnp.float32),
            pltpu.VMEM_SHARED((CR, D), jnp.float32),
            pltpu.VMEM((CR, D), jnp.float32),
            pltpu.VMEM((CR, D), jnp.float32),
            pltpu.VMEM((CR, D), jnp.float32),
            pltpu.VMEM((EPAD // 32,), jnp.int32),
            pltpu.VMEM((CR,), jnp.int32),
            pltpu.SemaphoreType.DMA,
        ],
        compiler_params=pltpu.CompilerParams(needs_layout_passes=False),
    )


def _seg_pass(table, src_hbm, dst_hbm, acc, srcv, dstv, r0, r1, sem0, sem1):
    sid = lax.axis_index("s")
    for blk in range(CH16 // CB):
        pltpu.sync_copy(src_hbm.at[sid, pl.ds(blk * CB, CB)], srcv)
        pltpu.sync_copy(dst_hbm.at[sid, pl.ds(blk * CB, CB)], dstv)
        pltpu.async_copy(table.at[srcv.at[0]], r0, sem0)

        @pl.loop(0, CB // 2 - 1)
        def _(j2):
            a = j2 * 2
            pltpu.async_copy(table.at[srcv.at[a + 1]], r1, sem1)
            pltpu.make_async_copy(table.at[srcv.at[a]], r0, sem0).wait()
            pltpu.sync_copy(r0, acc.at[dstv.at[a]], add=True)
            pltpu.async_copy(table.at[srcv.at[a + 2]], r0, sem0)
            pltpu.make_async_copy(table.at[srcv.at[a + 1]], r1, sem1).wait()
            pltpu.sync_copy(r1, acc.at[dstv.at[a + 1]], add=True)

        a = CB - 2
        pltpu.async_copy(table.at[srcv.at[a + 1]], r1, sem1)
        pltpu.make_async_copy(table.at[srcv.at[a]], r0, sem0).wait()
        pltpu.sync_copy(r0, acc.at[dstv.at[a]], add=True)
        pltpu.make_async_copy(table.at[srcv.at[a + 1]], r1, sem1).wait()
        pltpu.sync_copy(r1, acc.at[dstv.at[a + 1]], add=True)


def _layer_body(xpl, xph, xgl, xgh,
                s_isa, d_isa, s_rel, d_rel, s_rev, d_rev, z64, out,
                table, acc, srcv, dstv, r0, r1, sem0, sem1):
    cid = lax.axis_index("c")
    sid = lax.axis_index("s")
    rsh = sid * RT

    def load_table(xlo, xhi):
        @pl.when(jnp.logical_and(cid == 0, sid < 10))
        def _():
            pltpu.sync_copy(xlo.at[pl.ds(sid * 1000, 1000)],
                            table.at[pl.ds(sid * 1000, 1000)])

        @pl.when(jnp.logical_and(cid == 1, sid < 10))
        def _():
            pltpu.sync_copy(xhi.at[pl.ds(sid * 1000, 1000)],
                            table.at[pl.ds(sid * 1000, 1000)])

    passes = [(xpl, xph, s_isa, d_isa, True), (xpl, xph, s_rel, d_rel, False),
              (xgl, xgh, s_rev, d_rev, True)]
    for e, (xlo, xhi, src_e, dst_e, load) in enumerate(passes):
        pltpu.sync_copy(z64.at[pl.ds(rsh, RT)], acc.at[pl.ds(rsh, RT)])
        if load:
            load_table(xlo, xhi)
        plsc.subcore_barrier()
        _seg_pass(table, src_e, dst_e, acc, srcv, dstv, r0, r1, sem0, sem1)
        plsc.subcore_barrier()
        off = (e * NC + cid) * RX + rsh
        pltpu.sync_copy(acc.at[pl.ds(rsh, RT)], out.at[pl.ds(off, RT)])


@functools.lru_cache(maxsize=None)
def _layer_kernel():
    return pl.kernel(
        _layer_body,
        out_type=jax.ShapeDtypeStruct((3 * NC * RX, HD), jnp.float32),
        mesh=_mesh(),
        scratch_types=[
            pltpu.VMEM_SHARED((N_P, HD), jnp.float32),
            pltpu.VMEM_SHARED((RX, HD), jnp.float32),
            pltpu.VMEM((CB, CW), jnp.int32),
            pltpu.VMEM((CB, CW), jnp.int32),
            pltpu.VMEM((CW, HD), jnp.float32),
            pltpu.VMEM((CW, HD), jnp.float32),
            pltpu.SemaphoreType.DMA,
            pltpu.SemaphoreType.DMA,
        ],
        compiler_params=pltpu.CompilerParams(use_tc_tiling_on_sc=False),
    )


BR = 1000


def _update_body(act, sums, cnt, xpl, xph, xgl, xgh, wl, wr, b2,
                 opl, oph, ogl, ogh):
    c = jnp.maximum(cnt[:, :3] + cnt[:, 3:], 1.0)
    dot = functools.partial(jnp.dot, preferred_element_type=jnp.float32)
    m_isa = jnp.concatenate([sums[0, 0], sums[0, 1]], axis=-1) / c[:, 0:1]
    m_rel = jnp.concatenate([sums[1, 0], sums[1, 1]], axis=-1) / c[:, 1:2]
    m_rev = jnp.concatenate([sums[2, 0], sums[2, 1]], axis=-1) / c[:, 2:3]
    xp = jnp.concatenate([xpl[...], xph[...]], axis=-1)
    xg = jnp.concatenate([xgl[...], xgh[...]], axis=-1)
    p = (dot(m_isa, wl[0]) + dot(m_rev, wl[2])
         + dot(xp, wr[0]) + dot(xp, wr[2]) + b2[0])
    g = dot(m_rel, wl[1]) + dot(xg, wr[1]) + b2[1]
    if act:
        p = jnp.where(p >= 0, p, 0.01 * p)
        g = jnp.where(g >= 0, g, 0.01 * g)
    opl[...] = p[:, :HD]
    oph[...] = p[:, HD:]
    ogl[...] = g[:, :HD]
    ogh[...] = g[:, HD:]


def _update(act, sums, cnt, xpl, xph, xgl, xgh, wl, wr, b2):
    grid = N_P // BR
    half = jax.ShapeDtypeStruct((N_P, HD), jnp.float32)
    return pl.pallas_call(
        functools.partial(_update_body, act),
        grid=(grid,),
        in_specs=[
            pl.BlockSpec((3, NC, BR, HD), lambda i: (0, 0, i, 0)),
            pl.BlockSpec((BR, 6), lambda i: (i, 0)),
            pl.BlockSpec((BR, HD), lambda i: (i, 0)),
            pl.BlockSpec((BR, HD), lambda i: (i, 0)),
            pl.BlockSpec((BR, HD), lambda i: (i, 0)),
            pl.BlockSpec((BR, HD), lambda i: (i, 0)),
            pl.BlockSpec((3, D, D), lambda i: (0, 0, 0)),
            pl.BlockSpec((3, D, D), lambda i: (0, 0, 0)),
            pl.BlockSpec((2, 1, D), lambda i: (0, 0, 0)),
        ],
        out_specs=[pl.BlockSpec((BR, HD), lambda i: (i, 0))] * 4,
        out_shape=[half, half, half, half],
    )(sums, cnt, xpl, xph, xgl, xgh, wl, wr, b2)


def _decode_gather_body(xpl, xph, xgl, xgh, pidx, gidx, outp, outg,
                        tab, iv, r0, r1, sem0, sem1):
    cid = lax.axis_index("c")
    sid = lax.axis_index("s")

    def load(xlo, xhi):
        @pl.when(sid < 10)
        def _():
            pltpu.sync_copy(xlo.at[pl.ds(sid * 1000, 1000)],
                            tab.at[0, pl.ds(sid * 1000, 1000)])
            pltpu.sync_copy(xhi.at[pl.ds(sid * 1000, 1000)],
                            tab.at[1, pl.ds(sid * 1000, 1000)])

    def side(idx_hbm, out_hbm):
        pltpu.sync_copy(idx_hbm.at[sid], iv)
        base = sid * DCH * CW
        pltpu.async_copy(tab.at[0].at[iv.at[0]], r0, sem0)

        @pl.loop(0, DCH - 1)
        def _(j):
            off = base + j * CW
            pltpu.async_copy(tab.at[1].at[iv.at[j]], r1, sem1)
            pltpu.make_async_copy(tab.at[0].at[iv.at[j]], r0, sem0).wait()
            pltpu.sync_copy(r0, out_hbm.at[pl.ds(off, CW)])
            pltpu.async_copy(tab.at[0].at[iv.at[j + 1]], r0, sem0)
            pltpu.make_async_copy(tab.at[1].at[iv.at[j]], r1, sem1).wait()
            pltpu.sync_copy(r1, out_hbm.at[pl.ds(ELPAD + off, CW)])

        off = base + (DCH - 1) * CW
        j = DCH - 1
        pltpu.async_copy(tab.at[1].at[iv.at[j]], r1, sem1)
        pltpu.make_async_copy(tab.at[0].at[iv.at[j]], r0, sem0).wait()
        pltpu.sync_copy(r0, out_hbm.at[pl.ds(off, CW)])
        pltpu.make_async_copy(tab.at[1].at[iv.at[j]], r1, sem1).wait()
        pltpu.sync_copy(r1, out_hbm.at[pl.ds(ELPAD + off, CW)])

    @pl.when(cid == 0)
    def _():
        load(xpl, xph)
        plsc.subcore_barrier()
        side(pidx, outp)

    @pl.when(cid == 1)
    def _():
        load(xgl, xgh)
        plsc.subcore_barrier()
        side(gidx, outg)


@functools.lru_cache(maxsize=None)
def _decode_gather():
    return pl.kernel(
        _decode_gather_body,
        out_type=[jax.ShapeDtypeStruct((2 * ELPAD, HD), jnp.float32),
                  jax.ShapeDtypeStruct((2 * ELPAD, HD), jnp.float32)],
        mesh=_mesh(),
        scratch_types=[
            pltpu.VMEM_SHARED((2, N_P, HD), jnp.float32),
            pltpu.VMEM((DCH, CW), jnp.int32),
            pltpu.VMEM((CW, HD), jnp.float32),
            pltpu.VMEM((CW, HD), jnp.float32),
            pltpu.SemaphoreType.DMA,
            pltpu.SemaphoreType.DMA,
        ],
        compiler_params=pltpu.CompilerParams(use_tc_tiling_on_sc=False),
    )


DB = 4096


def _decode_dot_body(efp, efg, out):
    s = jnp.sum(efp[0] * efg[0] + efp[1] * efg[1], axis=-1)
    out[...] = 1.0 / (1.0 + jnp.exp(-s))


def _decode_dot(efp, efg):
    return pl.pallas_call(
        _decode_dot_body,
        grid=(ELPAD // DB,),
        in_specs=[
            pl.BlockSpec((2, DB, HD), lambda i: (0, i, 0)),
            pl.BlockSpec((2, DB, HD), lambda i: (0, i, 0)),
        ],
        out_specs=pl.BlockSpec((DB,), lambda i: (i,)),
        out_shape=jax.ShapeDtypeStruct((ELPAD,), jnp.float32),
    )(efp, efg)


def kernel(x_pheno, x_gene, edge_index_isa, edge_index_rel, edge_index_rev,
           edge_label_index, Wl, Wr, b):
    f32 = jnp.float32
    s_isa, d_isa = _prep_edges(edge_index_isa)
    s_rel, d_rel = _prep_edges(edge_index_rel)
    s_rev, d_rev = _prep_edges(edge_index_rev)

    riota = jnp.arange(CR, dtype=jnp.int32)
    z128 = jnp.zeros((R, D), f32)
    z64 = jnp.zeros((RX, HD), f32)

    cnt = _counts_kernel()(d_isa.reshape(32, EPAD // 32),
                           d_rel.reshape(32, EPAD // 32),
                           d_rev.reshape(32, EPAD // 32), riota, z128)
    cnt = cnt.reshape(NC * 3, CR * D).T

    b2 = jnp.stack([b[:, 0] + b[:, 2], b[:, 1]], axis=1)
    b2 = b2[:, :, None, :]

    xpl, xph = x_pheno[:, :HD], x_pheno[:, HD:]
    xgl, xgh = x_gene[:, :HD], x_gene[:, HD:]
    for l in range(L):
        sums = _layer_kernel()(xpl, xph, xgl, xgh,
                               s_isa, d_isa, s_rel, d_rel, s_rev, d_rev, z64)
        xpl, xph, xgl, xgh = _update(l < L - 1, sums.reshape(3, NC, RX, HD),
                                     cnt, xpl, xph, xgl, xgh,
                                     Wl[l], Wr[l], b2[l])

    pidx = edge_label_index[0].astype(jnp.int32)
    gidx = edge_label_index[1].astype(jnp.int32)
    pad = ELPAD - EL
    pidx = jnp.concatenate([pidx, jnp.zeros((pad,), jnp.int32)])
    gidx = jnp.concatenate([gidx, jnp.zeros((pad,), jnp.int32)])
    pidx = pidx.reshape(NS, DCH, CW)
    gidx = gidx.reshape(NS, DCH, CW)

    efp, efg = _decode_gather()(xpl, xph, xgl, xgh, pidx, gidx)
    out = _decode_dot(efp.reshape(2, ELPAD, HD), efg.reshape(2, ELPAD, HD))
    return out[:EL]


# speedup vs baseline: 6.2241x; 1.1638x over previous
import functools

import jax
import jax.numpy as jnp
from jax import lax
from jax.experimental import pallas as pl
from jax.experimental.pallas import tpu as pltpu
from jax.experimental.pallas import tpu_sc as plsc

N_P = 10000
N_G = 10000
D = 128
HD = 64
E = 320000
EL = 100000
L = 3

NC = 2
NS = 16

R = 10240
CR = 80
DUMMY = 10016
RX = 10112
RT = RX // NS

CW = 128
CB = 40
EPAD = 327680
CH16 = 160

DCH = 50
ELPAD = NS * DCH * CW
_SC_PARAMS = None


@functools.lru_cache(maxsize=None)
def _mesh():
    return plsc.VectorSubcoreMesh(core_axis_name="c", subcore_axis_name="s",
                                  num_cores=NC, num_subcores=NS)


def _prep_edges(e_idx):
    src = e_idx[0].astype(jnp.int32)
    dst = e_idx[1].astype(jnp.int32)
    src = jnp.concatenate([src, jnp.zeros((EPAD - E,), jnp.int32)])
    dst = jnp.concatenate([dst, jnp.full((EPAD - E,), DUMMY, jnp.int32)])
    return src.reshape(NS, CH16, CW), dst.reshape(NS, CH16, CW)


def _counts_body(d_isa, d_rel, d_rev, riota, z128, out,
                 sh0, sh1, sh2, c0, c1, c2, dstv, riov, sem):
    cid = lax.axis_index("c")
    sid = lax.axis_index("s")
    wid = cid * NS + sid
    shs = (sh0, sh1, sh2)
    cnts = (c0, c1, c2)
    dsts = (d_isa, d_rel, d_rev)
    for a in cnts:
        pltpu.sync_copy(z128.at[pl.ds(0, CR)], a)

    @pl.when(sid == 0)
    def _():
        for a in shs:
            pltpu.sync_copy(z128.at[pl.ds(0, CR)], a)

    pltpu.sync_copy(riota, riov)
    plsc.subcore_barrier()
    ones16 = jnp.ones((16,), jnp.float32)
    lane = lax.iota(jnp.int32, 16)
    for e in range(3):
        pltpu.sync_copy(dsts[e].at[wid], dstv)

        @pl.loop(0, R // 16)
        def _(g):
            idx = dstv[pl.ds(g * 16, 16)]
            hi = lax.shift_right_logical(idx, 7)
            lo = lax.bitwise_and(idx, 127)
            for k in range(16):
                plsc.addupdate_scatter(cnts[e], [hi, lo], ones16,
                                       mask=lane == k)

    for e in range(3):
        pltpu.sync_copy(cnts[e], shs[e].at[riov], add=True)
    plsc.subcore_barrier()

    @pl.when(sid == 0)
    def _():
        for e in range(3):
            pltpu.sync_copy(shs[e], out.at[pl.ds((cid * 3 + e) * CR, CR)])


@functools.lru_cache(maxsize=None)
def _counts_kernel():
    return pl.kernel(
        _counts_body,
        out_type=jax.ShapeDtypeStruct((NC * 3 * CR, D), jnp.float32),
        mesh=_mesh(),
        scratch_types=[
            pltpu.VMEM_SHARED((CR, D), jnp.float32),
            pltpu.VMEM_SHARED((CR, D), jnp.float32),
            pltpu.VMEM_SHARED((CR, D), jnp.float32),
            pltpu.VMEM((CR, D), jnp.float32),
            pltpu.VMEM((CR, D), jnp.float32),
            pltpu.VMEM((CR, D), jnp.float32),
            pltpu.VMEM((EPAD // 32,), jnp.int32),
            pltpu.VMEM((CR,), jnp.int32),
            pltpu.SemaphoreType.DMA,
        ],
        compiler_params=pltpu.CompilerParams(needs_layout_passes=False),
    )


def _seg_pass(table, src_hbm, dst_hbm, acc, srcv, dstv, rr, gs, ss):
    sid = lax.axis_index("s")

    def g_issue(j, k):
        pltpu.async_copy(table.at[srcv.at[j]], rr[k], gs[k])

    def g_wait(j, k):
        pltpu.make_async_copy(table.at[srcv.at[j]], rr[k], gs[k]).wait()

    def s_issue(j, k):
        pltpu.async_copy(rr[k], acc.at[dstv.at[j]], ss[k], add=True)

    def s_wait(k):
        pltpu.make_async_copy(rr[k], acc.at[dstv.at[0]], ss[k]).wait()

    for blk in range(CH16 // CB):
        pltpu.sync_copy(src_hbm.at[sid, pl.ds(blk * CB, CB)], srcv)
        pltpu.sync_copy(dst_hbm.at[sid, pl.ds(blk * CB, CB)], dstv)
        g_issue(0, 0)
        g_issue(1, 1)
        for j in (0, 1):
            g_wait(j, j)
            s_issue(j, j)
            g_issue(j + 2, j + 2)

        @pl.loop(0, (CB - 4) // 4)
        def _(J):
            for k in range(4):
                j = 2 + J * 4 + k
                kc = (2 + k) % 4
                kn = k
                g_wait(j, kc)
                s_issue(j, kc)
                s_wait(kn)
                g_issue(j + 2, kn)

        for j in (CB - 2, CB - 1):
            kc = j % 4
            g_wait(j, kc)
            s_issue(j, kc)
        for k in range(4):
            s_wait(k)


def _layer_body(xpl, xph, xgl, xgh,
                s_isa, d_isa, s_rel, d_rel, s_rev, d_rev, z64, out,
                table, acc, srcv, dstv, r0, r1, r2, r3,
                g0, g1, g2, g3, ss0, ss1, ss2, ss3):
    rr = (r0, r1, r2, r3)
    gs = (g0, g1, g2, g3)
    ss = (ss0, ss1, ss2, ss3)
    cid = lax.axis_index("c")
    sid = lax.axis_index("s")
    rsh = sid * RT

    def load_table(xlo, xhi):
        @pl.when(jnp.logical_and(cid == 0, sid < 10))
        def _():
            pltpu.sync_copy(xlo.at[pl.ds(sid * 1000, 1000)],
                            table.at[pl.ds(sid * 1000, 1000)])

        @pl.when(jnp.logical_and(cid == 1, sid < 10))
        def _():
            pltpu.sync_copy(xhi.at[pl.ds(sid * 1000, 1000)],
                            table.at[pl.ds(sid * 1000, 1000)])

    passes = [(xpl, xph, s_isa, d_isa, True), (xpl, xph, s_rel, d_rel, False),
              (xgl, xgh, s_rev, d_rev, True)]
    for e, (xlo, xhi, src_e, dst_e, load) in enumerate(passes):
        pltpu.sync_copy(z64.at[pl.ds(rsh, RT)], acc.at[pl.ds(rsh, RT)])
        if load:
            load_table(xlo, xhi)
        plsc.subcore_barrier()
        _seg_pass(table, src_e, dst_e, acc, srcv, dstv, rr, gs, ss)
        plsc.subcore_barrier()
        off = (e * NC + cid) * RX + rsh
        pltpu.sync_copy(acc.at[pl.ds(rsh, RT)], out.at[pl.ds(off, RT)])


@functools.lru_cache(maxsize=None)
def _layer_kernel():
    return pl.kernel(
        _layer_body,
        out_type=jax.ShapeDtypeStruct((3 * NC * RX, HD), jnp.float32),
        mesh=_mesh(),
        scratch_types=[
            pltpu.VMEM_SHARED((N_P, HD), jnp.float32),
            pltpu.VMEM_SHARED((RX, HD), jnp.float32),
            pltpu.VMEM((CB, CW), jnp.int32),
            pltpu.VMEM((CB, CW), jnp.int32),
            pltpu.VMEM((CW, HD), jnp.float32),
            pltpu.VMEM((CW, HD), jnp.float32),
            pltpu.VMEM((CW, HD), jnp.float32),
            pltpu.VMEM((CW, HD), jnp.float32),
            pltpu.SemaphoreType.DMA,
            pltpu.SemaphoreType.DMA,
            pltpu.SemaphoreType.DMA,
            pltpu.SemaphoreType.DMA,
            pltpu.SemaphoreType.DMA,
            pltpu.SemaphoreType.DMA,
            pltpu.SemaphoreType.DMA,
            pltpu.SemaphoreType.DMA,
        ],
        compiler_params=pltpu.CompilerParams(use_tc_tiling_on_sc=False),
    )


BR = 1000


def _update_body(act, sums, cnt, xpl, xph, xgl, xgh, wl, wr, b2,
                 opl, oph, ogl, ogh):
    c = jnp.maximum(cnt[:, :3] + cnt[:, 3:], 1.0)
    dot = functools.partial(jnp.dot, preferred_element_type=jnp.float32)
    m_isa = jnp.concatenate([sums[0, 0], sums[0, 1]], axis=-1) / c[:, 0:1]
    m_rel = jnp.concatenate([sums[1, 0], sums[1, 1]], axis=-1) / c[:, 1:2]
    m_rev = jnp.concatenate([sums[2, 0], sums[2, 1]], axis=-1) / c[:, 2:3]
    xp = jnp.concatenate([xpl[...], xph[...]], axis=-1)
    xg = jnp.concatenate([xgl[...], xgh[...]], axis=-1)
    p = (dot(m_isa, wl[0]) + dot(m_rev, wl[2])
         + dot(xp, wr[0]) + dot(xp, wr[2]) + b2[0])
    g = dot(m_rel, wl[1]) + dot(xg, wr[1]) + b2[1]
    if act:
        p = jnp.where(p >= 0, p, 0.01 * p)
        g = jnp.where(g >= 0, g, 0.01 * g)
    opl[...] = p[:, :HD]
    oph[...] = p[:, HD:]
    ogl[...] = g[:, :HD]
    ogh[...] = g[:, HD:]


def _update(act, sums, cnt, xpl, xph, xgl, xgh, wl, wr, b2):
    grid = N_P // BR
    half = jax.ShapeDtypeStruct((N_P, HD), jnp.float32)
    return pl.pallas_call(
        functools.partial(_update_body, act),
        grid=(grid,),
        in_specs=[
            pl.BlockSpec((3, NC, BR, HD), lambda i: (0, 0, i, 0)),
            pl.BlockSpec((BR, 6), lambda i: (i, 0)),
            pl.BlockSpec((BR, HD), lambda i: (i, 0)),
            pl.BlockSpec((BR, HD), lambda i: (i, 0)),
            pl.BlockSpec((BR, HD), lambda i: (i, 0)),
            pl.BlockSpec((BR, HD), lambda i: (i, 0)),
            pl.BlockSpec((3, D, D), lambda i: (0, 0, 0)),
            pl.BlockSpec((3, D, D), lambda i: (0, 0, 0)),
            pl.BlockSpec((2, 1, D), lambda i: (0, 0, 0)),
        ],
        out_specs=[pl.BlockSpec((BR, HD), lambda i: (i, 0))] * 4,
        out_shape=[half, half, half, half],
    )(sums, cnt, xpl, xph, xgl, xgh, wl, wr, b2)


def _decode_gather_body(xpl, xph, xgl, xgh, pidx, gidx, outp, outg,
                        tab, iv, r0, r1, sem0, sem1):
    cid = lax.axis_index("c")
    sid = lax.axis_index("s")

    def load(xlo, xhi):
        @pl.when(sid < 10)
        def _():
            pltpu.sync_copy(xlo.at[pl.ds(sid * 1000, 1000)],
                            tab.at[0, pl.ds(sid * 1000, 1000)])
            pltpu.sync_copy(xhi.at[pl.ds(sid * 1000, 1000)],
                            tab.at[1, pl.ds(sid * 1000, 1000)])

    def side(idx_hbm, out_hbm):
        pltpu.sync_copy(idx_hbm.at[sid], iv)
        base = sid * DCH * CW
        pltpu.async_copy(tab.at[0].at[iv.at[0]], r0, sem0)

        @pl.loop(0, DCH - 1)
        def _(j):
            off = base + j * CW
            pltpu.async_copy(tab.at[1].at[iv.at[j]], r1, sem1)
            pltpu.make_async_copy(tab.at[0].at[iv.at[j]], r0, sem0).wait()
            pltpu.sync_copy(r0, out_hbm.at[pl.ds(off, CW)])
            pltpu.async_copy(tab.at[0].at[iv.at[j + 1]], r0, sem0)
            pltpu.make_async_copy(tab.at[1].at[iv.at[j]], r1, sem1).wait()
            pltpu.sync_copy(r1, out_hbm.at[pl.ds(ELPAD + off, CW)])

        off = base + (DCH - 1) * CW
        j = DCH - 1
        pltpu.async_copy(tab.at[1].at[iv.at[j]], r1, sem1)
        pltpu.make_async_copy(tab.at[0].at[iv.at[j]], r0, sem0).wait()
        pltpu.sync_copy(r0, out_hbm.at[pl.ds(off, CW)])
        pltpu.make_async_copy(tab.at[1].at[iv.at[j]], r1, sem1).wait()
        pltpu.sync_copy(r1, out_hbm.at[pl.ds(ELPAD + off, CW)])

    @pl.when(cid == 0)
    def _():
        load(xpl, xph)
        plsc.subcore_barrier()
        side(pidx, outp)

    @pl.when(cid == 1)
    def _():
        load(xgl, xgh)
        plsc.subcore_barrier()
        side(gidx, outg)


@functools.lru_cache(maxsize=None)
def _decode_gather():
    return pl.kernel(
        _decode_gather_body,
        out_type=[jax.ShapeDtypeStruct((2 * ELPAD, HD), jnp.float32),
                  jax.ShapeDtypeStruct((2 * ELPAD, HD), jnp.float32)],
        mesh=_mesh(),
        scratch_types=[
            pltpu.VMEM_SHARED((2, N_P, HD), jnp.float32),
            pltpu.VMEM((DCH, CW), jnp.int32),
            pltpu.VMEM((CW, HD), jnp.float32),
            pltpu.VMEM((CW, HD), jnp.float32),
            pltpu.SemaphoreType.DMA,
            pltpu.SemaphoreType.DMA,
        ],
        compiler_params=pltpu.CompilerParams(use_tc_tiling_on_sc=False),
    )


DB = 4096


def _decode_dot_body(efp, efg, out):
    s = jnp.sum(efp[0] * efg[0] + efp[1] * efg[1], axis=-1)
    out[...] = 1.0 / (1.0 + jnp.exp(-s))


def _decode_dot(efp, efg):
    return pl.pallas_call(
        _decode_dot_body,
        grid=(ELPAD // DB,),
        in_specs=[
            pl.BlockSpec((2, DB, HD), lambda i: (0, i, 0)),
            pl.BlockSpec((2, DB, HD), lambda i: (0, i, 0)),
        ],
        out_specs=pl.BlockSpec((DB,), lambda i: (i,)),
        out_shape=jax.ShapeDtypeStruct((ELPAD,), jnp.float32),
    )(efp, efg)


def kernel(x_pheno, x_gene, edge_index_isa, edge_index_rel, edge_index_rev,
           edge_label_index, Wl, Wr, b):
    f32 = jnp.float32
    s_isa, d_isa = _prep_edges(edge_index_isa)
    s_rel, d_rel = _prep_edges(edge_index_rel)
    s_rev, d_rev = _prep_edges(edge_index_rev)

    riota = jnp.arange(CR, dtype=jnp.int32)
    z128 = jnp.zeros((R, D), f32)
    z64 = jnp.zeros((RX, HD), f32)

    cnt = _counts_kernel()(d_isa.reshape(32, EPAD // 32),
                           d_rel.reshape(32, EPAD // 32),
                           d_rev.reshape(32, EPAD // 32), riota, z128)
    cnt = cnt.reshape(NC * 3, CR * D).T

    b2 = jnp.stack([b[:, 0] + b[:, 2], b[:, 1]], axis=1)
    b2 = b2[:, :, None, :]

    xpl, xph = x_pheno[:, :HD], x_pheno[:, HD:]
    xgl, xgh = x_gene[:, :HD], x_gene[:, HD:]
    for l in range(L):
        sums = _layer_kernel()(xpl, xph, xgl, xgh,
                               s_isa, d_isa, s_rel, d_rel, s_rev, d_rev, z64)
        xpl, xph, xgl, xgh = _update(l < L - 1, sums.reshape(3, NC, RX, HD),
                                     cnt, xpl, xph, xgl, xgh,
                                     Wl[l], Wr[l], b2[l])

    pidx = edge_label_index[0].astype(jnp.int32)
    gidx = edge_label_index[1].astype(jnp.int32)
    pad = ELPAD - EL
    pidx = jnp.concatenate([pidx, jnp.zeros((pad,), jnp.int32)])
    gidx = jnp.concatenate([gidx, jnp.zeros((pad,), jnp.int32)])
    pidx = pidx.reshape(NS, DCH, CW)
    gidx = gidx.reshape(NS, DCH, CW)

    efp, efg = _decode_gather()(xpl, xph, xgl, xgh, pidx, gidx)
    out = _decode_dot(efp.reshape(2, ELPAD, HD), efg.reshape(2, ELPAD, HD))
    return out[:EL]
